# baseline (speedup 1.0000x reference)
import functools

import jax
import jax.numpy as jnp
import numpy as np
from jax import lax
from jax.experimental import pallas as pl
from jax.experimental.pallas import tpu as pltpu
from jax.experimental.pallas import tpu_sc as plsc

N = 16384
NB = 1000
SMOOTH = 0.1
COLD = np.float32(SMOOTH / NB)
HOT = np.float32(np.float32(1.0 - SMOOTH) + COLD)

LANES = 16
CHUNK_ROWS = 64
CHUNK_WORDS = CHUNK_ROWS * NB


def _build_sc_call(num_cores: int, num_subcores: int):
    num_workers = num_cores * num_subcores
    rows_per_w = N // num_workers
    n_chunks = rows_per_w // CHUNK_ROWS
    mesh = plsc.VectorSubcoreMesh(
        core_axis_name="c", subcore_axis_name="s",
        num_cores=num_cores, num_subcores=num_subcores)

    @functools.partial(
        pl.kernel,
        out_type=jax.ShapeDtypeStruct((N * NB,), jnp.float32),
        mesh=mesh,
        scratch_types=[
            pltpu.VMEM((rows_per_w,), jnp.int32),
            pltpu.VMEM((CHUNK_WORDS,), jnp.float32),
            pltpu.VMEM((CHUNK_WORDS,), jnp.float32),
            pltpu.SemaphoreType.DMA,
            pltpu.SemaphoreType.DMA,
        ],
        compiler_params=pltpu.CompilerParams(needs_layout_passes=False),
    )
    def sc_kernel(x_hbm, fill_hbm, out_hbm, lab_v, buf0, buf1, sem0, sem1):
        wid = lax.axis_index("s") * num_cores + lax.axis_index("c")
        base_row = wid * rows_per_w
        pltpu.sync_copy(x_hbm.at[pl.ds(base_row, rows_per_w)], lab_v)
        bufs, sems = [buf0, buf1], [sem0, sem1]
        pltpu.sync_copy(fill_hbm, buf0)
        pltpu.sync_copy(fill_hbm, buf1)

        hot = jnp.full((LANES,), HOT, jnp.float32)
        cold = jnp.full((LANES,), COLD, jnp.float32)
        lane_off = lax.iota(jnp.int32, LANES) * NB

        def flat_idx(ci, j):
            labs = lab_v[pl.ds(ci * CHUNK_ROWS + j * LANES, LANES)]
            return labs + (j * LANES * NB) + lane_off

        pending = [None, None]
        for ci in range(n_chunks):
            b = ci % 2
            if pending[b] is not None:
                pending[b].wait()
                for j in range(CHUNK_ROWS // LANES):
                    plsc.store_scatter(bufs[b], [flat_idx(ci - 2, j)], cold)
            for j in range(CHUNK_ROWS // LANES):
                plsc.store_scatter(bufs[b], [flat_idx(ci, j)], hot)
            out_base = (base_row + ci * CHUNK_ROWS) * NB
            pending[b] = pltpu.async_copy(
                bufs[b], out_hbm.at[pl.ds(out_base, CHUNK_WORDS)], sems[b])
        pending[0].wait()
        pending[1].wait()

    return sc_kernel


def kernel(x):
    info = plsc.get_sparse_core_info()
    sc_call = _build_sc_call(info.num_cores, info.num_subcores)
    fill = jnp.full((CHUNK_WORDS,), COLD, jnp.float32)
    out_flat = sc_call(x.astype(jnp.int32), fill)
    return out_flat.reshape(N, NB)


# speedup vs baseline: 1.5481x; 1.5481x over previous
import functools

import jax
import jax.numpy as jnp
import numpy as np
from jax import lax
from jax.experimental import pallas as pl
from jax.experimental.pallas import tpu as pltpu
from jax.experimental.pallas import tpu_sc as plsc

N = 16384
NB = 1000
SMOOTH = 0.1
COLD = np.float32(SMOOTH / NB)
HOT = np.float32(np.float32(1.0 - SMOOTH) + COLD)

LANES = 16
CHUNK_ROWS = 32
CHUNK_WORDS = CHUNK_ROWS * NB


def _build_sc_call(num_cores: int, num_subcores: int):
    num_workers = num_cores * num_subcores
    rows_per_w = N // num_workers
    n_chunks = rows_per_w // CHUNK_ROWS
    mesh = plsc.VectorSubcoreMesh(
        core_axis_name="c", subcore_axis_name="s",
        num_cores=num_cores, num_subcores=num_subcores)

    @functools.partial(
        pl.kernel,
        out_type=jax.ShapeDtypeStruct((N, NB), jnp.float32),
        mesh=mesh,
        scratch_types=[
            pltpu.VMEM((rows_per_w,), jnp.int32),
            pltpu.VMEM((CHUNK_ROWS, NB), jnp.float32),
            pltpu.VMEM((CHUNK_ROWS, NB), jnp.float32),
            pltpu.SemaphoreType.DMA,
            pltpu.SemaphoreType.DMA,
        ],
        compiler_params=pltpu.CompilerParams(needs_layout_passes=False),
    )
    def sc_kernel(x_hbm, fill_hbm, out_hbm, lab_v, buf0, buf1, sem0, sem1):
        wid = lax.axis_index("s") * num_cores + lax.axis_index("c")
        base_row = wid * rows_per_w
        pltpu.sync_copy(x_hbm.at[pl.ds(base_row, rows_per_w)], lab_v)
        bufs, sems = [buf0, buf1], [sem0, sem1]
        pltpu.sync_copy(fill_hbm, buf0)
        pltpu.sync_copy(fill_hbm, buf1)

        hot = jnp.full((LANES,), HOT, jnp.float32)
        cold = jnp.full((LANES,), COLD, jnp.float32)
        lane_rows = lax.iota(jnp.int32, LANES)

        def hot_idx(ci, j):
            labs = lab_v[pl.ds(ci * CHUNK_ROWS + j * LANES, LANES)]
            return [lane_rows + j * LANES, labs]

        pending = [None, None]
        for ci in range(n_chunks):
            b = ci % 2
            if pending[b] is not None:
                pending[b].wait()
                for j in range(CHUNK_ROWS // LANES):
                    plsc.store_scatter(bufs[b], hot_idx(ci - 2, j), cold)
            for j in range(CHUNK_ROWS // LANES):
                plsc.store_scatter(bufs[b], hot_idx(ci, j), hot)
            row0 = base_row + ci * CHUNK_ROWS
            pending[b] = pltpu.async_copy(
                bufs[b], out_hbm.at[pl.ds(row0, CHUNK_ROWS), :], sems[b])
        pending[0].wait()
        pending[1].wait()

    return sc_kernel


def kernel(x):
    info = plsc.get_sparse_core_info()
    sc_call = _build_sc_call(info.num_cores, info.num_subcores)
    fill = jnp.full((CHUNK_ROWS, NB), COLD, jnp.float32)
    return sc_call(x.astype(jnp.int32), fill)


# speedup vs baseline: 3.2611x; 2.1065x over previous
import functools

import jax
import jax.numpy as jnp
import numpy as np
from jax import lax
from jax.experimental import pallas as pl
from jax.experimental.pallas import tpu as pltpu
from jax.experimental.pallas import tpu_sc as plsc

N = 16384
NB = 1000
SMOOTH = 0.1
COLD = np.float32(SMOOTH / NB)
HOT = np.float32(np.float32(1.0 - SMOOTH) + COLD)

LANES = 16
CHUNK_COLS = 128


def _build_sc_call(num_cores: int, num_subcores: int):
    num_workers = num_cores * num_subcores
    cols_per_w = N // num_workers
    n_chunks = cols_per_w // CHUNK_COLS
    mesh = plsc.VectorSubcoreMesh(
        core_axis_name="c", subcore_axis_name="s",
        num_cores=num_cores, num_subcores=num_subcores)

    @functools.partial(
        pl.kernel,
        out_type=jax.ShapeDtypeStruct((NB, N), jnp.float32),
        mesh=mesh,
        scratch_types=[
            pltpu.VMEM((cols_per_w,), jnp.int32),
            pltpu.VMEM((NB, CHUNK_COLS), jnp.float32),
        ],
        compiler_params=pltpu.CompilerParams(needs_layout_passes=False),
    )
    def sc_kernel(x_hbm, fill_hbm, out_hbm, lab_v, buf_v):
        wid = lax.axis_index("s") * num_cores + lax.axis_index("c")
        base_col = wid * cols_per_w
        pltpu.sync_copy(x_hbm.at[pl.ds(base_col, cols_per_w)], lab_v)
        pltpu.sync_copy(fill_hbm, buf_v)

        hot = jnp.full((LANES,), HOT, jnp.float32)
        cold = jnp.full((LANES,), COLD, jnp.float32)
        lane = lax.iota(jnp.int32, LANES)

        def hot_idx(ci, j):
            labs = lab_v[pl.ds(ci * CHUNK_COLS + j * LANES, LANES)]
            return [labs, lane + j * LANES]

        def chunk_body(ci, carry):
            for j in range(CHUNK_COLS // LANES):
                plsc.store_scatter(buf_v, hot_idx(ci, j), hot)
            col0 = base_col + ci * CHUNK_COLS
            pltpu.sync_copy(buf_v, out_hbm.at[:, pl.ds(col0, CHUNK_COLS)])
            for j in range(CHUNK_COLS // LANES):
                plsc.store_scatter(buf_v, hot_idx(ci, j), cold)
            return carry

        lax.fori_loop(0, n_chunks, chunk_body, 0)

    return sc_kernel


def kernel(x):
    info = plsc.get_sparse_core_info()
    sc_call = _build_sc_call(info.num_cores, info.num_subcores)
    fill = jnp.full((NB, CHUNK_COLS), COLD, jnp.float32)
    return sc_call(x.astype(jnp.int32), fill).T
